# Initial kernel scaffold; baseline (speedup 1.0000x reference)
import jax
import jax.numpy as jnp
from jax.experimental import pallas as pl


def kernel(x_p, x_d, edge_attr_p, edge_attr_d, edge_index_p, x_p_batch, nn0_W1, nn0_b1, nn0_W2, nn0_b2, nn1_W1, nn1_b1, nn1_W2, nn1_b2, root0, bias0, root1, bias1, lin0_W, lin0_b, lin1_W, lin1_b):
    raise NotImplementedError("write your pallas kernel here")


# speedup vs baseline: 2.6349x; 2.6349x over previous
import functools

import jax
import jax.numpy as jnp
from jax import lax
from jax.experimental import pallas as pl
from jax.experimental.pallas import tpu as pltpu
from jax.experimental.pallas import tpu_sc as plsc

_N = 10000
_E = 160000
_F_IN = 32
_EMB = 16
_NG = 64

_CHUNK = 128
_NCH = _E // _CHUNK
_NW = 32
_ITERS = (_NCH + _NW - 1) // _NW


def _sc_mesh():
    return plsc.VectorSubcoreMesh(core_axis_name="c", subcore_axis_name="s")


@functools.lru_cache(maxsize=None)
def _make_gather(feat):
    @functools.partial(
        pl.kernel,
        out_type=jax.ShapeDtypeStruct((_E, feat), jnp.float32),
        mesh=_sc_mesh(),
        scratch_types=[
            pltpu.VMEM((_CHUNK,), jnp.int32),
            pltpu.VMEM((_CHUNK, feat), jnp.float32),
            pltpu.SemaphoreType.DMA,
        ],
        compiler_params=pltpu.CompilerParams(use_tc_tiling_on_sc=False),
    )
    def gather(table_hbm, idx_hbm, out_hbm, idx_v, rows_v, sem):
        wid = lax.axis_index("s") * 2 + lax.axis_index("c")

        def body(i, carry):
            cid = wid + i * _NW

            @pl.when(cid < _NCH)
            def _():
                pltpu.sync_copy(idx_hbm.at[cid], idx_v)
                pltpu.async_copy(table_hbm.at[idx_v], rows_v, sem).wait()
                pltpu.sync_copy(rows_v, out_hbm.at[pl.ds(cid * _CHUNK, _CHUNK)])

            return carry

        lax.fori_loop(0, _ITERS, body, 0)

    return gather


@functools.lru_cache(maxsize=None)
def _make_scatter_add():
    @functools.partial(
        pl.kernel,
        out_type=jax.ShapeDtypeStruct((2, _N, _EMB), jnp.float32),
        mesh=_sc_mesh(),
        scratch_types=[
            pltpu.VMEM((_CHUNK,), jnp.int32),
            pltpu.VMEM((_CHUNK, _EMB), jnp.float32),
            pltpu.VMEM_SHARED((_N, _EMB), jnp.float32),
        ],
        compiler_params=pltpu.CompilerParams(use_tc_tiling_on_sc=False),
    )
    def scatter_add(msg_hbm, idx_hbm, zeros_hbm, out_hbm, idx_v, msg_v, acc_sh):
        c = lax.axis_index("c")
        s = lax.axis_index("s")

        @pl.when(s == 0)
        def _():
            pltpu.sync_copy(zeros_hbm, acc_sh)

        plsc.subcore_barrier()

        def body(i, carry):
            cid = (s * 2 + c) + i * _NW

            @pl.when(cid < _NCH)
            def _():
                pltpu.sync_copy(idx_hbm.at[cid], idx_v)
                pltpu.sync_copy(msg_hbm.at[pl.ds(cid * _CHUNK, _CHUNK)], msg_v)
                pltpu.sync_copy(msg_v, acc_sh.at[idx_v], add=True)

            return carry

        lax.fori_loop(0, _ITERS, body, 0)
        plsc.subcore_barrier()

        @pl.when(s < 10)
        def _():
            pltpu.sync_copy(
                acc_sh.at[pl.ds(s * 1000, 1000)],
                out_hbm.at[c].at[pl.ds(s * 1000, 1000)],
            )

    return scatter_add


def _msg_body(ea_ref, xg_ref, w1_ref, b1_ref, w2t_ref, r_ref, s_ref, b2_ref,
              out_ref):
    h = jnp.maximum(
        lax.dot(ea_ref[...], w1_ref[...], preferred_element_type=jnp.float32)
        + b1_ref[...], 0.0)
    hb = lax.dot(h, r_ref[...], preferred_element_type=jnp.float32)
    z = lax.dot(xg_ref[...], w2t_ref[...], preferred_element_type=jnp.float32)
    out_ref[...] = (
        lax.dot(z * hb, s_ref[...], preferred_element_type=jnp.float32)
        + lax.dot(xg_ref[...], b2_ref[...], preferred_element_type=jnp.float32))


def _msg_call(ea, xg, w1, b1, w2t, r_mat, s_mat, b2_mat):
    feat = xg.shape[1]
    tile = 2000
    full = lambda shape: pl.BlockSpec(shape, lambda i: (0, 0))
    return pl.pallas_call(
        _msg_body,
        grid=(_E // tile,),
        in_specs=[
            pl.BlockSpec((tile, 16), lambda i: (i, 0)),
            pl.BlockSpec((tile, feat), lambda i: (i, 0)),
            full((16, 16)),
            full((1, 16)),
            full((feat, 256)),
            full((16, 256)),
            full((256, _EMB)),
            full((feat, _EMB)),
        ],
        out_specs=pl.BlockSpec((tile, _EMB), lambda i: (i, 0)),
        out_shape=jax.ShapeDtypeStruct((_E, _EMB), jnp.float32),
    )(ea, xg, w1, b1.reshape(1, 16), w2t, r_mat, s_mat, b2_mat)


def _update_body(p0_ref, p1_ref, x_ref, root_ref, bias_ref, out_ref):
    agg = p0_ref[...] + p1_ref[...]
    out_ref[...] = jnp.maximum(
        agg + lax.dot(x_ref[...], root_ref[...],
                      preferred_element_type=jnp.float32) + bias_ref[...], 0.0)


def _update_call(p0, p1, x, root, bias):
    feat = x.shape[1]
    tile = 1000
    full = lambda shape: pl.BlockSpec(shape, lambda i: (0, 0))
    return pl.pallas_call(
        _update_body,
        grid=(_N // tile,),
        in_specs=[
            pl.BlockSpec((tile, _EMB), lambda i: (i, 0)),
            pl.BlockSpec((tile, _EMB), lambda i: (i, 0)),
            pl.BlockSpec((tile, feat), lambda i: (i, 0)),
            full((feat, _EMB)),
            full((1, _EMB)),
        ],
        out_specs=pl.BlockSpec((tile, _EMB), lambda i: (i, 0)),
        out_shape=jax.ShapeDtypeStruct((_N, _EMB), jnp.float32),
    )(p0, p1, x, root, bias.reshape(1, _EMB))


def _final_body(p0_ref, p1_ref, x_ref, root_ref, bias_ref, batch_ref,
                l0w_ref, l0b_ref, l1w_ref, l1b_ref, out_ref, pool_ref):
    x2 = jnp.maximum(
        p0_ref[...] + p1_ref[...]
        + lax.dot(x_ref[...], root_ref[...],
                  preferred_element_type=jnp.float32) + bias_ref[...], 0.0)
    batch = batch_ref[...]

    def body(g, carry):
        m = jnp.where(batch == g, x2, -jnp.inf)
        pool_ref[pl.ds(g, 1), :] = jnp.max(m, axis=0, keepdims=True)
        return carry

    lax.fori_loop(0, _NG, body, 0)
    hidden = lax.dot(pool_ref[...], l0w_ref[...],
                     preferred_element_type=jnp.float32) + l0b_ref[...]
    out_ref[...] = lax.dot(hidden, l1w_ref[...],
                           preferred_element_type=jnp.float32) + l1b_ref[...]


def _final_call(p0, p1, x1, root, bias, batch, l0w, l0b, l1w, l1b):
    return pl.pallas_call(
        _final_body,
        out_shape=jax.ShapeDtypeStruct((_NG, 1), jnp.float32),
        scratch_shapes=[pltpu.VMEM((_NG, _EMB), jnp.float32)],
    )(p0, p1, x1, root, bias.reshape(1, _EMB), batch.reshape(_N, 1),
      l0w, l0b.reshape(1, _EMB), l1w, l1b.reshape(1, 1))


def _prep_w2(w2, feat):
    return w2.reshape(16, feat, _EMB).transpose(1, 0, 2).reshape(feat, 16 * _EMB)


@jax.jit
def kernel(x_p, x_d, edge_attr_p, edge_attr_d, edge_index_p, x_p_batch,
           nn0_W1, nn0_b1, nn0_W2, nn0_b2,
           nn1_W1, nn1_b1, nn1_W2, nn1_b2,
           root0, bias0, root1, bias1,
           lin0_W, lin0_b, lin1_W, lin1_b):
    src2d = edge_index_p[0].reshape(_NCH, _CHUNK)
    dst2d = edge_index_p[1].reshape(_NCH, _CHUNK)

    kk = jnp.arange(16 * _EMB) // _EMB
    oo = jnp.arange(16 * _EMB) % _EMB
    r_mat = (kk[None, :] == jnp.arange(16)[:, None]).astype(jnp.float32)
    s_mat = (oo[:, None] == jnp.arange(_EMB)[None, :]).astype(jnp.float32)
    zeros_n = jnp.zeros((_N, _EMB), jnp.float32)

    w2t0 = _prep_w2(nn0_W2, _F_IN)
    b20 = nn0_b2.reshape(_F_IN, _EMB)
    w2t1 = _prep_w2(nn1_W2, _EMB)
    b21 = nn1_b2.reshape(_EMB, _EMB)

    scatter_add = _make_scatter_add()

    xg0 = _make_gather(_F_IN)(x_p, src2d)
    msg0 = _msg_call(edge_attr_p, xg0, nn0_W1, nn0_b1, w2t0, r_mat, s_mat, b20)
    parts0 = scatter_add(msg0, dst2d, zeros_n)
    x1 = _update_call(parts0[0], parts0[1], x_p, root0, bias0)

    xg1 = _make_gather(_EMB)(x1, src2d)
    msg1 = _msg_call(edge_attr_p, xg1, nn1_W1, nn1_b1, w2t1, r_mat, s_mat, b21)
    parts1 = scatter_add(msg1, dst2d, zeros_n)

    return _final_call(parts1[0], parts1[1], x1, root1, bias1, x_p_batch,
                       lin0_W, lin0_b, lin1_W, lin1_b)
